# baseline (speedup 1.0000x reference)
import functools

import jax
import jax.numpy as jnp
from jax import lax
from jax.experimental import pallas as pl
from jax.experimental.pallas import tpu as pltpu
from jax.experimental.pallas import tpu_sc as plsc

ALPHA_R = 0.65 * 0.1

B = 128
L = 8192
NLANE = 16
SEG = L // NLANE
NW = 32
ROWS_PER_W = B // NW


def _make_sc_kernel():
    mesh = plsc.VectorSubcoreMesh(core_axis_name="c", subcore_axis_name="s")

    @functools.partial(
        pl.kernel,
        mesh=mesh,
        compiler_params=pltpu.CompilerParams(needs_layout_passes=False),
        out_type=jax.ShapeDtypeStruct((NW, NLANE), jnp.float32),
        scratch_types=[
            pltpu.VMEM((ROWS_PER_W * 2 * L,), jnp.float32),
            pltpu.VMEM((ROWS_PER_W * L,), jnp.int32),
            pltpu.VMEM((L,), jnp.float32),
            pltpu.VMEM((NLANE,), jnp.float32),
        ],
    )
    def sc_body(out_hbm, lab_hbm, nci_hbm, res_hbm, obuf, lbuf, nbuf, rbuf):
        wid = lax.axis_index("s") * 2 + lax.axis_index("c")
        row0 = wid * ROWS_PER_W
        pltpu.sync_copy(out_hbm.at[pl.ds(row0 * 2 * L, ROWS_PER_W * 2 * L)], obuf)
        pltpu.sync_copy(lab_hbm.at[pl.ds(row0 * L, ROWS_PER_W * L)], lbuf)
        pltpu.sync_copy(nci_hbm, nbuf)

        lanes = lax.iota(jnp.int32, NLANE)
        lane_o = lanes * (2 * SEG)
        lane_l = lanes * SEG
        zerov = jnp.zeros((NLANE,), jnp.float32)
        zeroi = jnp.zeros((NLANE,), jnp.int32)

        acc = jnp.float32(0.0)
        for rr in range(ROWS_PER_W):
            obase = lane_o + rr * (2 * L)
            lbase = lane_l + rr * L

            def body(i, carry):
                run, rec, hasz = carry
                oi = obase + 2 * i
                c0 = plsc.load_gather(obuf, [oi])
                c1 = plsc.load_gather(obuf, [oi + 1])
                lab = plsc.load_gather(lbuf, [lbase + i])
                ncv = nbuf[pl.ds(i * NLANE, NLANE)]
                zero = c0 >= c1
                rec = jnp.where(zero, run, rec)
                hasz = jnp.where(zero, 1, hasz)
                run = run + c1 * jnp.where(lab == 1, ncv, ALPHA_R)
                return run, rec, hasz

            run, rec, hasz = lax.fori_loop(
                0, SEG, body, (zerov, zerov, zeroi), unroll=4
            )
            mlane = jnp.max(jnp.where(hasz == 1, lanes, -1))
            m2 = jnp.where(mlane < 0, NLANE, mlane)
            acc = acc + jnp.sum(
                jnp.where(lanes < m2, run, jnp.where(lanes == m2, rec, 0.0))
            )

        resv = jnp.where(lanes == 0, acc * (1.0 / B), 0.0)
        rbuf[...] = resv
        pltpu.sync_copy(rbuf, res_hbm.at[wid])

    return sc_body


_sc_kernel = _make_sc_kernel()


@jax.jit
def kernel(output, labels):
    out1d = output.reshape(B * 2 * L)
    lab1d = labels.reshape(B * L)
    j = jnp.arange(L, dtype=jnp.float32)
    nci = -3.6 / jnp.log2(j + 2.0)
    nci_perm = nci.reshape(NLANE, SEG).T.reshape(-1)
    partials = _sc_kernel(out1d, lab1d, nci_perm)
    return jnp.sum(partials)


# speedup vs baseline: 1.0006x; 1.0006x over previous
import functools

import jax
import jax.numpy as jnp
from jax import lax
from jax.experimental import pallas as pl
from jax.experimental.pallas import tpu as pltpu
from jax.experimental.pallas import tpu_sc as plsc

ALPHA_R = 0.65 * 0.1

B = 128
L = 8192
NLANE = 16
SEG = L // NLANE
NW = 32
ROWS_PER_W = B // NW

OW = 2 * L // 128
LW = L // 128


def _make_sc_kernel():
    mesh = plsc.VectorSubcoreMesh(core_axis_name="c", subcore_axis_name="s")

    @functools.partial(
        pl.kernel,
        mesh=mesh,
        compiler_params=pltpu.CompilerParams(
            needs_layout_passes=False, use_tc_tiling_on_sc=True
        ),
        out_type=jax.ShapeDtypeStruct((NW, 128), jnp.float32),
        scratch_types=[
            pltpu.VMEM((ROWS_PER_W * OW, 128), jnp.float32),
            pltpu.VMEM((ROWS_PER_W * LW, 128), jnp.int32),
            pltpu.VMEM((L // 128, 128), jnp.float32),
            pltpu.VMEM((1, 128), jnp.float32),
        ],
    )
    def sc_body(out_hbm, lab_hbm, nci_hbm, res_hbm, obuf, lbuf, nbuf, rbuf):
        wid = lax.axis_index("s") * 2 + lax.axis_index("c")
        pltpu.sync_copy(out_hbm.at[pl.ds(wid * ROWS_PER_W * OW, ROWS_PER_W * OW)], obuf)
        pltpu.sync_copy(lab_hbm.at[pl.ds(wid * ROWS_PER_W * LW, ROWS_PER_W * LW)], lbuf)
        pltpu.sync_copy(nci_hbm, nbuf)

        lanes = lax.iota(jnp.int32, NLANE)
        lane_o = lanes * (2 * SEG)
        lane_l = lanes * SEG
        zerov = jnp.zeros((NLANE,), jnp.float32)
        zeroi = jnp.zeros((NLANE,), jnp.int32)

        acc = jnp.float32(0.0)
        for rr in range(ROWS_PER_W):
            obase = lane_o + rr * (2 * L)
            lbase = lane_l + rr * L

            def body(i, carry):
                run, rec, hasz = carry
                oi = obase + 2 * i
                li = lbase + i
                c0 = plsc.load_gather(obuf, [oi >> 7, oi & 127])
                c1 = plsc.load_gather(obuf, [(oi + 1) >> 7, (oi + 1) & 127])
                lab = plsc.load_gather(lbuf, [li >> 7, li & 127])
                ncv = nbuf[i >> 3, pl.ds((i & 7) * NLANE, NLANE)]
                zero = c0 >= c1
                rec = jnp.where(zero, run, rec)
                hasz = jnp.where(zero, 1, hasz)
                run = run + c1 * jnp.where(lab == 1, ncv, ALPHA_R)
                return run, rec, hasz

            run, rec, hasz = lax.fori_loop(
                0, SEG, body, (zerov, zerov, zeroi), unroll=4
            )
            mlane = jnp.max(jnp.where(hasz == 1, lanes, -1))
            m2 = jnp.where(mlane < 0, NLANE, mlane)
            acc = acc + jnp.sum(
                jnp.where(lanes < m2, run, jnp.where(lanes == m2, rec, 0.0))
            )

        for p in range(8):
            rbuf[0, pl.ds(p * NLANE, NLANE)] = jnp.where(
                lanes == 0, jnp.where(p == 0, acc * (1.0 / B), 0.0), 0.0
            )
        pltpu.sync_copy(rbuf, res_hbm.at[pl.ds(wid, 1)])

    return sc_body


_sc_kernel = _make_sc_kernel()


@jax.jit
def kernel(output, labels):
    out2d = output.reshape(B * OW, 128)
    lab2d = labels.reshape(B * LW, 128)
    j = jnp.arange(L, dtype=jnp.float32)
    nci = -3.6 / jnp.log2(j + 2.0)
    nci_perm = nci.reshape(NLANE, SEG).T.reshape(L // 128, 128)
    partials = _sc_kernel(out2d, lab2d, nci_perm)
    return jnp.sum(partials)


# speedup vs baseline: 20.7827x; 20.7706x over previous
import functools

import jax
import jax.numpy as jnp
from jax import lax
from jax.experimental import pallas as pl
from jax.experimental.pallas import tpu as pltpu
from jax.experimental.pallas import tpu_sc as plsc

ALPHA_R = 0.65 * 0.1

B = 128
L = 8192
NLANE = 16
SEG = L // NLANE
NW = 32
ROWS_PER_W = B // NW
NT = L // 128


def _make_sc_kernel():
    mesh = plsc.VectorSubcoreMesh(core_axis_name="c", subcore_axis_name="s")

    @functools.partial(
        pl.kernel,
        mesh=mesh,
        compiler_params=pltpu.CompilerParams(
            needs_layout_passes=False, use_tc_tiling_on_sc=True
        ),
        out_type=jax.ShapeDtypeStruct((NW, 128), jnp.float32),
        scratch_types=[
            pltpu.VMEM((ROWS_PER_W * 2 * NT, 128), jnp.float32),
            pltpu.VMEM((ROWS_PER_W * NT, 128), jnp.int32),
            pltpu.VMEM((L // 128, 128), jnp.float32),
            pltpu.VMEM((1, 128), jnp.float32),
            pltpu.SemaphoreType.DMA,
        ],
    )
    def sc_body(out_hbm, lab_hbm, nci_hbm, res_hbm, obuf, lbuf, nbuf, rbuf, sem):
        wid = lax.axis_index("s") * 2 + lax.axis_index("c")
        cp_o = pltpu.make_async_copy(
            out_hbm.at[pl.ds(wid * (ROWS_PER_W * 2 * NT), ROWS_PER_W * 2 * NT)],
            obuf,
            sem,
        )
        cp_o.start()
        bt = wid >> 1
        s0 = (wid & 1) * ROWS_PER_W
        copies = []
        for t in range(NT):
            cp = pltpu.make_async_copy(
                lab_hbm.at[pl.ds(bt * (8 * NT) + t * 8 + s0, ROWS_PER_W)],
                lbuf.at[pl.ds(t * ROWS_PER_W, ROWS_PER_W)],
                sem,
            )
            cp.start()
            copies.append(cp)
        cp_n = pltpu.make_async_copy(nci_hbm, nbuf, sem)
        cp_n.start()
        cp_o.wait()
        for cp in copies:
            cp.wait()
        cp_n.wait()

        lanes = lax.iota(jnp.int32, NLANE)
        lane8 = lanes * 8
        lane16 = lanes * 16
        zerov = jnp.zeros((NLANE,), jnp.float32)
        zeroi = jnp.zeros((NLANE,), jnp.int32)

        acc = jnp.float32(0.0)
        for rr in range(ROWS_PER_W):
            orow0 = lane8 + rr * (2 * NT)
            lrow0 = lane16 + rr

            def body(i, carry):
                run, rec, hasz = carry
                th = i >> 7
                col = i & 127
                colv = jnp.broadcast_to(col, (NLANE,))
                c0r = orow0 + th * 2
                lr = lrow0 + th * 4
                c0 = plsc.load_gather(obuf, [c0r, colv])
                c1 = plsc.load_gather(obuf, [c0r + 1, colv])
                lab = plsc.load_gather(lbuf, [lr, colv])
                ncv = nbuf[i >> 3, pl.ds((i & 7) * NLANE, NLANE)]
                zero = c0 >= c1
                rec = jnp.where(zero, run, rec)
                hasz = jnp.where(zero, 1, hasz)
                run = run + c1 * jnp.where(lab == 1, ncv, ALPHA_R)
                return run, rec, hasz

            run, rec, hasz = lax.fori_loop(
                0, SEG, body, (zerov, zerov, zeroi), unroll=4
            )
            mlane = jnp.max(jnp.where(hasz == 1, lanes, -1))
            m2 = jnp.where(mlane < 0, NLANE, mlane)
            acc = acc + jnp.sum(
                jnp.where(lanes < m2, run, jnp.where(lanes == m2, rec, 0.0))
            )

        for p in range(8):
            rbuf[0, pl.ds(p * NLANE, NLANE)] = jnp.where(
                lanes == 0, jnp.where(p == 0, acc * (1.0 / B), 0.0), 0.0
            )
        pltpu.sync_copy(rbuf, res_hbm.at[pl.ds(wid, 1)])

    return sc_body


_sc_kernel = _make_sc_kernel()


@jax.jit
def kernel(output, labels):
    out_v = output.reshape(B, NT, 128, 2).transpose(0, 1, 3, 2).reshape(B * 2 * NT, 128)
    lab_v = labels.reshape(B // 8, 8, NT, 128).transpose(0, 2, 1, 3).reshape(B * NT, 128)
    j = jnp.arange(L, dtype=jnp.float32)
    nci = -3.6 / jnp.log2(j + 2.0)
    nci_perm = nci.reshape(NLANE, SEG).T.reshape(L // 128, 128)
    partials = _sc_kernel(out_v, lab_v, nci_perm)
    return jnp.sum(partials)


# speedup vs baseline: 42.3972x; 2.0400x over previous
import functools

import jax
import jax.numpy as jnp
from jax import lax
from jax.experimental import pallas as pl
from jax.experimental.pallas import tpu as pltpu
from jax.experimental.pallas import tpu_sc as plsc

ALPHA_R = 0.65 * 0.1

B = 128
L = 8192
NLANE = 16
NCHUNK = L // NLANE
NW = 32
ROWS_PER_W = B // NW
NT = L // 128


def _make_sc_kernel():
    mesh = plsc.VectorSubcoreMesh(core_axis_name="c", subcore_axis_name="s")

    @functools.partial(
        pl.kernel,
        mesh=mesh,
        compiler_params=pltpu.CompilerParams(
            needs_layout_passes=False, use_tc_tiling_on_sc=True
        ),
        out_type=jax.ShapeDtypeStruct((NW, 128), jnp.float32),
        scratch_types=[
            pltpu.VMEM((ROWS_PER_W * 2 * NT, 128), jnp.float32),
            pltpu.VMEM((ROWS_PER_W * NT, 128), jnp.int32),
            pltpu.VMEM((L // 128, 128), jnp.float32),
            pltpu.VMEM((1, 128), jnp.float32),
            pltpu.SemaphoreType.DMA,
        ],
    )
    def sc_body(out_hbm, lab_hbm, nci_hbm, res_hbm, obuf, lbuf, nbuf, rbuf, sem):
        wid = lax.axis_index("s") * 2 + lax.axis_index("c")
        cp_o = pltpu.make_async_copy(
            out_hbm.at[pl.ds(wid * (ROWS_PER_W * 2 * NT), ROWS_PER_W * 2 * NT)],
            obuf,
            sem,
        )
        cp_o.start()
        bt = wid >> 1
        s0 = (wid & 1) * ROWS_PER_W
        copies = []
        for t in range(NT):
            cp = pltpu.make_async_copy(
                lab_hbm.at[pl.ds(bt * (8 * NT) + t * 8 + s0, ROWS_PER_W)],
                lbuf.at[pl.ds(t * ROWS_PER_W, ROWS_PER_W)],
                sem,
            )
            cp.start()
            copies.append(cp)
        cp_n = pltpu.make_async_copy(nci_hbm, nbuf, sem)
        cp_n.start()
        cp_o.wait()
        for cp in copies:
            cp.wait()
        cp_n.wait()

        lanes = lax.iota(jnp.int32, NLANE)
        negv = jnp.full((NLANE,), -1, jnp.int32)
        zerov = jnp.zeros((NLANE,), jnp.float32)

        acc = jnp.float32(0.0)
        for rr in range(ROWS_PER_W):
            orow0 = rr * (2 * NT)

            def find_body(k, zpos):
                tr = orow0 + (k >> 3) * 2
                cs = (k & 7) * NLANE
                c0 = obuf[tr, pl.ds(cs, NLANE)]
                c1 = obuf[tr + 1, pl.ds(cs, NLANE)]
                jv = k * NLANE + lanes
                return jnp.where(c0 >= c1, jv, zpos)

            zpos = lax.fori_loop(0, NCHUNK, find_body, negv, unroll=8)
            last_zero = jnp.max(zpos)
            cut = jnp.where(last_zero < 0, L, last_zero)

            def sum_body(k, va):
                tr = orow0 + (k >> 3) * 2
                lr = (k >> 3) * ROWS_PER_W + rr
                cs = (k & 7) * NLANE
                c1 = obuf[tr + 1, pl.ds(cs, NLANE)]
                lab = lbuf[lr, pl.ds(cs, NLANE)]
                ncv = nbuf[k >> 3, pl.ds(cs, NLANE)]
                jv = k * NLANE + lanes
                v = c1 * jnp.where(lab == 1, ncv, ALPHA_R)
                return va + jnp.where(jv < cut, v, 0.0)

            va = lax.fori_loop(0, NCHUNK, sum_body, zerov, unroll=8)
            acc = acc + jnp.sum(va)

        for p in range(8):
            rbuf[0, pl.ds(p * NLANE, NLANE)] = jnp.where(
                lanes == 0, jnp.where(p == 0, acc * (1.0 / B), 0.0), 0.0
            )
        pltpu.sync_copy(rbuf, res_hbm.at[pl.ds(wid, 1)])

    return sc_body


_sc_kernel = _make_sc_kernel()


@jax.jit
def kernel(output, labels):
    out_v = output.reshape(B, NT, 128, 2).transpose(0, 1, 3, 2).reshape(B * 2 * NT, 128)
    lab_v = labels.reshape(B // 8, 8, NT, 128).transpose(0, 2, 1, 3).reshape(B * NT, 128)
    j = jnp.arange(L, dtype=jnp.float32)
    nci = (-3.6 / jnp.log2(j + 2.0)).reshape(L // 128, 128)
    partials = _sc_kernel(out_v, lab_v, nci)
    return jnp.sum(partials)


# speedup vs baseline: 45.0314x; 1.0621x over previous
import functools

import jax
import jax.numpy as jnp
import numpy as np
from jax import lax
from jax.experimental import pallas as pl
from jax.experimental.pallas import tpu as pltpu
from jax.experimental.pallas import tpu_sc as plsc

ALPHA_R = 0.65 * 0.1

B = 128
L = 8192
NLANE = 16
NCHUNK = L // NLANE
NW = 32
ROWS_PER_W = B // NW
NT = L // 128


def _make_sc_kernel():
    mesh = plsc.VectorSubcoreMesh(core_axis_name="c", subcore_axis_name="s")

    @functools.partial(
        pl.kernel,
        mesh=mesh,
        compiler_params=pltpu.CompilerParams(
            needs_layout_passes=False, use_tc_tiling_on_sc=True
        ),
        out_type=jax.ShapeDtypeStruct((NW, 128), jnp.float32),
        scratch_types=[
            pltpu.VMEM((ROWS_PER_W * 2 * NT, 128), jnp.float32),
            pltpu.VMEM((ROWS_PER_W * NT, 128), jnp.int32),
            pltpu.VMEM((L // 128, 128), jnp.float32),
            pltpu.VMEM((1, 128), jnp.float32),
            pltpu.SemaphoreType.DMA,
        ],
    )
    def sc_body(out_hbm, lab_hbm, nci_hbm, res_hbm, obuf, lbuf, nbuf, rbuf, sem):
        wid = lax.axis_index("s") * 2 + lax.axis_index("c")
        cp_o = pltpu.make_async_copy(
            out_hbm.at[pl.ds(wid * (ROWS_PER_W * 2 * NT), ROWS_PER_W * 2 * NT)],
            obuf,
            sem,
        )
        cp_o.start()
        bt = wid >> 1
        s0 = (wid & 1) * ROWS_PER_W
        copies = []
        for t in range(NT):
            cp = pltpu.make_async_copy(
                lab_hbm.at[pl.ds(bt * (8 * NT) + t * 8 + s0, ROWS_PER_W)],
                lbuf.at[pl.ds(t * ROWS_PER_W, ROWS_PER_W)],
                sem,
            )
            cp.start()
            copies.append(cp)
        cp_n = pltpu.make_async_copy(nci_hbm, nbuf, sem)
        cp_n.start()
        cp_o.wait()

        lanes = lax.iota(jnp.int32, NLANE)
        negv = jnp.full((NLANE,), -1, jnp.int32)
        zerov = jnp.zeros((NLANE,), jnp.float32)

        cuts = []
        for rr in range(ROWS_PER_W):
            orow0 = rr * (2 * NT)

            def find_body(k, zpos):
                tr = orow0 + (k >> 3) * 2
                cs = (k & 7) * NLANE
                c0 = obuf[tr, pl.ds(cs, NLANE)]
                c1 = obuf[tr + 1, pl.ds(cs, NLANE)]
                jv = k * NLANE + lanes
                return jnp.where(c0 >= c1, jv, zpos)

            zpos = lax.fori_loop(0, NCHUNK, find_body, negv, unroll=8)
            last_zero = jnp.max(zpos)
            cuts.append(jnp.where(last_zero < 0, L, last_zero))

        for cp in copies:
            cp.wait()
        cp_n.wait()

        acc = jnp.float32(0.0)
        for rr in range(ROWS_PER_W):
            orow0 = rr * (2 * NT)
            cut = cuts[rr]

            def sum_body(k, va):
                tr = orow0 + (k >> 3) * 2
                lr = (k >> 3) * ROWS_PER_W + rr
                cs = (k & 7) * NLANE
                c1 = obuf[tr + 1, pl.ds(cs, NLANE)]
                lab = lbuf[lr, pl.ds(cs, NLANE)]
                ncv = nbuf[k >> 3, pl.ds(cs, NLANE)]
                jv = k * NLANE + lanes
                v = c1 * jnp.where(lab == 1, ncv, ALPHA_R)
                return va + jnp.where(jv < cut, v, 0.0)

            va = lax.fori_loop(0, NCHUNK, sum_body, zerov, unroll=8)
            acc = acc + jnp.sum(va)

        for p in range(8):
            rbuf[0, pl.ds(p * NLANE, NLANE)] = jnp.where(
                lanes == 0, jnp.where(p == 0, acc * (1.0 / B), 0.0), 0.0
            )
        pltpu.sync_copy(rbuf, res_hbm.at[pl.ds(wid, 1)])

    return sc_body


_sc_kernel = _make_sc_kernel()


@jax.jit
def kernel(output, labels):
    out_v = output.reshape(B, NT, 128, 2).transpose(0, 1, 3, 2).reshape(B * 2 * NT, 128)
    lab_v = labels.reshape(B // 8, 8, NT, 128).transpose(0, 2, 1, 3).reshape(B * NT, 128)
    j = np.arange(L, dtype=np.float32)
    nci = jnp.asarray((-3.6 / np.log2(j + 2.0)).reshape(L // 128, 128))
    partials = _sc_kernel(out_v, lab_v, nci)
    return jnp.sum(partials)


# speedup vs baseline: 47.4804x; 1.0544x over previous
import functools

import jax
import jax.numpy as jnp
import numpy as np
from jax import lax
from jax.experimental import pallas as pl
from jax.experimental.pallas import tpu as pltpu
from jax.experimental.pallas import tpu_sc as plsc

ALPHA_R = 0.65 * 0.1

B = 128
L = 8192
NLANE = 16
NCHUNK = L // NLANE
NW = 32
ROWS_PER_W = B // NW
NT = L // 128


def _make_sc_kernel():
    mesh = plsc.VectorSubcoreMesh(core_axis_name="c", subcore_axis_name="s")

    @functools.partial(
        pl.kernel,
        mesh=mesh,
        compiler_params=pltpu.CompilerParams(
            needs_layout_passes=False, use_tc_tiling_on_sc=True
        ),
        out_type=jax.ShapeDtypeStruct((NW, 128), jnp.float32),
        scratch_types=[
            pltpu.VMEM((ROWS_PER_W * 2 * NT, 128), jnp.float32),
            pltpu.VMEM((ROWS_PER_W * NT, 128), jnp.int32),
            pltpu.VMEM((L // 128, 128), jnp.float32),
            pltpu.VMEM((1, 128), jnp.float32),
            pltpu.SemaphoreType.DMA,
        ],
    )
    def sc_body(out_hbm, lab_hbm, nci_hbm, res_hbm, obuf, lbuf, nbuf, rbuf, sem):
        wid = lax.axis_index("s") * 2 + lax.axis_index("c")
        cp_o = pltpu.make_async_copy(
            out_hbm.at[pl.ds(wid * (ROWS_PER_W * 2 * NT), ROWS_PER_W * 2 * NT)],
            obuf,
            sem,
        )
        cp_o.start()
        bt = wid >> 1
        s0 = (wid & 1) * ROWS_PER_W
        copies = []
        for t in range(NT):
            cp = pltpu.make_async_copy(
                lab_hbm.at[pl.ds(bt * (8 * NT) + t * 8 + s0, ROWS_PER_W)],
                lbuf.at[pl.ds(t * ROWS_PER_W, ROWS_PER_W)],
                sem,
            )
            cp.start()
            copies.append(cp)
        cp_n = pltpu.make_async_copy(nci_hbm, nbuf, sem)
        cp_n.start()
        cp_o.wait()

        lanes = lax.iota(jnp.int32, NLANE)
        zerov = jnp.zeros((NLANE,), jnp.float32)

        cuts = []
        for rr in range(ROWS_PER_W):
            orow0 = rr * (2 * NT)

            def find_cond(state):
                k, last = state
                return jnp.logical_and(last < 0, k >= 0)

            def find_body(state):
                k, _ = state
                tr = orow0 + (k >> 3) * 2
                cs = (k & 7) * NLANE
                c0 = obuf[tr, pl.ds(cs, NLANE)]
                c1 = obuf[tr + 1, pl.ds(cs, NLANE)]
                jv = k * NLANE + lanes
                last = jnp.max(jnp.where(c0 >= c1, jv, -1))
                return k - 1, last

            _, last_zero = lax.while_loop(
                find_cond, find_body, (jnp.int32(NCHUNK - 1), jnp.int32(-1))
            )
            cuts.append(jnp.where(last_zero < 0, L, last_zero))

        for cp in copies:
            cp.wait()
        cp_n.wait()

        def sum_body(k, vas):
            th = k >> 3
            cs = (k & 7) * NLANE
            ncv = nbuf[th, pl.ds(cs, NLANE)]
            jv = k * NLANE + lanes
            out = []
            for rr in range(ROWS_PER_W):
                c1 = obuf[rr * (2 * NT) + th * 2 + 1, pl.ds(cs, NLANE)]
                lab = lbuf[th * ROWS_PER_W + rr, pl.ds(cs, NLANE)]
                v = c1 * jnp.where(lab == 1, ncv, ALPHA_R)
                out.append(vas[rr] + jnp.where(jv < cuts[rr], v, 0.0))
            return tuple(out)

        vas = lax.fori_loop(
            0, NCHUNK, sum_body, (zerov,) * ROWS_PER_W, unroll=2
        )
        acc = jnp.float32(0.0)
        for rr in range(ROWS_PER_W):
            acc = acc + jnp.sum(vas[rr])

        for p in range(8):
            rbuf[0, pl.ds(p * NLANE, NLANE)] = jnp.where(
                lanes == 0, jnp.where(p == 0, acc * (1.0 / B), 0.0), 0.0
            )
        pltpu.sync_copy(rbuf, res_hbm.at[pl.ds(wid, 1)])

    return sc_body


_sc_kernel = _make_sc_kernel()


@jax.jit
def kernel(output, labels):
    out_v = output.reshape(B, NT, 128, 2).transpose(0, 1, 3, 2).reshape(B * 2 * NT, 128)
    lab_v = labels.reshape(B // 8, 8, NT, 128).transpose(0, 2, 1, 3).reshape(B * NT, 128)
    j = np.arange(L, dtype=np.float32)
    nci = jnp.asarray((-3.6 / np.log2(j + 2.0)).reshape(L // 128, 128))
    partials = _sc_kernel(out_v, lab_v, nci)
    return jnp.sum(partials)
